# baseline (speedup 1.0000x reference)
import jax
import jax.numpy as jnp
from jax import lax
from jax.experimental import pallas as pl
from jax.experimental.pallas import tpu as pltpu

EXPERTS = 64
K = 8
DM = 2048
DH = 768
TM = 256
TB = 256


def _router_body(x_ref, wr_ref, sel_ref, wts_ref):
    x = x_ref[...]
    logits = lax.dot_general(x, wr_ref[...], (((1,), (1,)), ((), ())),
                             preferred_element_type=jnp.float32)
    m = jnp.max(logits, axis=-1, keepdims=True)
    p = jnp.exp(logits - m)
    p = p / jnp.sum(p, axis=-1, keepdims=True)
    iota = lax.broadcasted_iota(jnp.int32, p.shape, 1)
    cur = p
    sels, ws = [], []
    for _ in range(K):
        mk = jnp.max(cur, axis=-1, keepdims=True)
        idx = jnp.min(jnp.where(cur == mk, iota, EXPERTS), axis=-1, keepdims=True)
        sels.append(idx)
        ws.append(mk)
        cur = jnp.where(iota == idx, -1.0, cur)
    sel = jnp.concatenate(sels, axis=-1)
    w = jnp.concatenate(ws, axis=-1)
    sel_ref[...] = sel
    wts_ref[...] = w / jnp.sum(w, axis=-1, keepdims=True)


def _router(flat, Wr):
    t = flat.shape[0]
    return pl.pallas_call(
        _router_body,
        grid=(t // TB,),
        in_specs=[pl.BlockSpec((TB, DM), lambda i: (i, 0)),
                  pl.BlockSpec((EXPERTS, DM), lambda i: (0, 0))],
        out_specs=[pl.BlockSpec((TB, K), lambda i: (i, 0)),
                   pl.BlockSpec((TB, K), lambda i: (i, 0))],
        out_shape=[jax.ShapeDtypeStruct((t, K), jnp.int32),
                   jax.ShapeDtypeStruct((t, K), jnp.float32)],
    )(flat, Wr)


RB = 256


def _masks_of(sel):
    iota_e = lax.broadcasted_iota(jnp.int32, (sel.shape[0], EXPERTS), 1)
    return [(sel[:, k:k + 1] == iota_e) for k in range(K)]


def _binning_body(sel_ref, ppos_ref, tiles_ref, base_ref, rank_ref, n_tiles):
    p = pl.program_id(0)
    i = pl.program_id(1)
    sel = sel_ref[...]
    masks = _masks_of(sel)

    @pl.when(p == 0)
    def _pass0():
        @pl.when(i == 0)
        def _():
            base_ref[...] = jnp.zeros_like(base_ref)
        onehot = jnp.zeros((RB, EXPERTS), jnp.float32)
        for k in range(K):
            onehot = onehot + masks[k].astype(jnp.float32)
        r = lax.broadcasted_iota(jnp.int32, (RB, RB), 0)
        c = lax.broadcasted_iota(jnp.int32, (RB, RB), 1)
        tri = (r > c).astype(jnp.float32)
        excl = lax.dot_general(tri, onehot, (((1,), (0,)), ((), ())),
                               preferred_element_type=jnp.float32)
        base = base_ref[...]
        vals = excl + base
        cols = [jnp.sum(jnp.where(masks[k], vals, 0.0), axis=1, keepdims=True)
                for k in range(K)]
        rank_ref[pl.ds(i * RB, RB), :] = jnp.concatenate(cols, axis=-1)
        base_ref[...] = base + jnp.sum(onehot, axis=0, keepdims=True)

    @pl.when(p == 1)
    def _pass1():
        counts = base_ref[...]
        ps = jnp.floor((counts + (TM - 1)) * (1.0 / TM)) * TM
        r64 = lax.broadcasted_iota(jnp.int32, (EXPERTS, EXPERTS), 0)
        c64 = lax.broadcasted_iota(jnp.int32, (EXPERTS, EXPERTS), 1)
        triu = (r64 <= c64).astype(jnp.float32)
        cum = lax.dot_general(ps, triu, (((1,), (0,)), ((), ())),
                              preferred_element_type=jnp.float32)
        pad_off = cum - ps
        rank_blk = rank_ref[pl.ds(i * RB, RB), :]
        offs = [jnp.sum(jnp.where(masks[k], pad_off, 0.0), axis=1,
                        keepdims=True) for k in range(K)]
        ppos_ref[...] = (rank_blk
                         + jnp.concatenate(offs, axis=-1)).astype(jnp.int32)

        @pl.when(i == 0)
        def _tiles():
            iota_e = lax.broadcasted_iota(jnp.int32, (1, EXPERTS), 1)
            cum_last = jnp.sum(jnp.where(iota_e == EXPERTS - 1, cum, 0.0),
                               axis=1, keepdims=True)
            starts = lax.broadcasted_iota(
                jnp.int32, (n_tiles, EXPERTS), 0).astype(
                    jnp.float32) * TM
            te = jnp.sum((cum <= starts).astype(jnp.float32), axis=1,
                         keepdims=True)
            te = jnp.minimum(te, EXPERTS - 1)
            tile_i = lax.broadcasted_iota(
                jnp.int32, (n_tiles, 1), 0).astype(jnp.float32)
            tv = (tile_i * TM < cum_last).astype(jnp.float32)
            n_valid = cum_last * (1.0 / TM)
            ti = jnp.minimum(tile_i, n_valid - 1.0)
            tiles_ref[...] = jnp.concatenate([te, tv, ti],
                                             axis=-1).astype(jnp.int32)


def _binning(sel, n_tiles):
    t = sel.shape[0]
    body = lambda *a: _binning_body(*a, n_tiles)
    return pl.pallas_call(
        body,
        grid=(2, t // RB),
        in_specs=[pl.BlockSpec((RB, K), lambda p, i: (i, 0))],
        out_specs=[pl.BlockSpec((RB, K), lambda p, i: (i, 0)),
                   pl.BlockSpec((n_tiles, 3), lambda p, i: (0, 0))],
        out_shape=[jax.ShapeDtypeStruct((t, K), jnp.int32),
                   jax.ShapeDtypeStruct((n_tiles, 3), jnp.int32)],
        scratch_shapes=[pltpu.VMEM((1, EXPERTS), jnp.float32),
                        pltpu.VMEM((t, K), jnp.float32)],
        compiler_params=pltpu.CompilerParams(
            dimension_semantics=("arbitrary", "arbitrary")),
    )(sel)


def _gmm_body(meta_ref, x_ref, wg_ref, wu_ref, wd_ref, w_ref, o_ref):
    i = pl.program_id(0)

    @pl.when(meta_ref[i, 1] > 0)
    def _():
        x = x_ref[...]
        g = lax.dot_general(x, wg_ref[0].astype(jnp.bfloat16),
                            (((1,), (1,)), ((), ())),
                            preferred_element_type=jnp.float32)
        u = lax.dot_general(x, wu_ref[0].astype(jnp.bfloat16),
                            (((1,), (1,)), ((), ())),
                            preferred_element_type=jnp.float32)
        h = (g * lax.logistic(g)) * u
        o = lax.dot_general(h.astype(jnp.bfloat16),
                            wd_ref[0].astype(jnp.bfloat16),
                            (((1,), (1,)), ((), ())),
                            preferred_element_type=jnp.float32)
        o_ref[...] = (o * w_ref[...]).astype(jnp.bfloat16)


def _gmm(xs, Wg, Wu, Wd, w_padded, tiles, n_tiles):
    m_max = n_tiles * TM
    grid_spec = pltpu.PrefetchScalarGridSpec(
        num_scalar_prefetch=1,
        grid=(n_tiles,),
        in_specs=[
            pl.BlockSpec((TM, DM), lambda i, mt: (mt[i, 2], 0)),
            pl.BlockSpec((1, DH, DM), lambda i, mt: (mt[i, 0], 0, 0)),
            pl.BlockSpec((1, DH, DM), lambda i, mt: (mt[i, 0], 0, 0)),
            pl.BlockSpec((1, DM, DH), lambda i, mt: (mt[i, 0], 0, 0)),
            pl.BlockSpec((TM, 1), lambda i, mt: (mt[i, 2], 0)),
        ],
        out_specs=pl.BlockSpec((TM, DM), lambda i, mt: (mt[i, 2], 0)),
    )
    return pl.pallas_call(
        _gmm_body,
        grid_spec=grid_spec,
        out_shape=jax.ShapeDtypeStruct((m_max, DM), jnp.bfloat16),
        compiler_params=pltpu.CompilerParams(
            dimension_semantics=("arbitrary",)),
    )(tiles, xs, Wg, Wu, Wd, w_padded.reshape(m_max, 1))


def kernel(hidden_states, Wr, Wg, Wu, Wd):
    orig_shape = hidden_states.shape
    flat = hidden_states.reshape(-1, orig_shape[-1])
    t = flat.shape[0]
    a = t * K
    n_tiles = a // TM + EXPERTS
    m_max = n_tiles * TM

    sel, wts = _router(flat, Wr)

    ppos, tiles = _binning(sel, n_tiles)
    ppos_flat = ppos.reshape(-1)
    w_padded = jnp.zeros((m_max,), jnp.float32).at[ppos_flat].set(
        wts.reshape(-1))
    flat_bf = flat.astype(jnp.bfloat16)
    rows = jnp.broadcast_to(flat_bf[:, None, :], (t, K, DM)).reshape(a, DM)
    xs = jnp.zeros((m_max, DM), jnp.bfloat16).at[ppos_flat].set(rows)

    out_s = _gmm(xs, Wg, Wu, Wd, w_padded, tiles, n_tiles)
    out = jnp.sum(out_s[ppos], axis=1, dtype=jnp.float32)
    return out.reshape(orig_shape)


# speedup vs baseline: 1.0131x; 1.0131x over previous
import jax
import jax.numpy as jnp
from jax import lax
from jax.experimental import pallas as pl
from jax.experimental.pallas import tpu as pltpu

EXPERTS = 64
K = 8
DM = 2048
DH = 768
TM = 256
TB = 256


RB = 256


def _masks_of(sel):
    iota_e = lax.broadcasted_iota(jnp.int32, (sel.shape[0], EXPERTS), 1)
    return [(sel[:, k:k + 1] == iota_e) for k in range(K)]


def _binning_body(x_ref, wr_ref, wts_ref, ppos_ref, tiles_ref,
                  base_ref, sel_scr, wts_scr, rank_ref, n_tiles):
    p = pl.program_id(0)
    i = pl.program_id(1)

    @pl.when(p == 0)
    def _pass0():
        @pl.when(i == 0)
        def _():
            base_ref[...] = jnp.zeros_like(base_ref)
        x = x_ref[...]
        logits = lax.dot_general(x, wr_ref[...], (((1,), (1,)), ((), ())),
                                 preferred_element_type=jnp.float32)
        m = jnp.max(logits, axis=-1, keepdims=True)
        prob = jnp.exp(logits - m)
        prob = prob / jnp.sum(prob, axis=-1, keepdims=True)
        iota = lax.broadcasted_iota(jnp.int32, prob.shape, 1)
        cur = prob
        sels, ws = [], []
        for _ in range(K):
            mk = jnp.max(cur, axis=-1, keepdims=True)
            idx = jnp.min(jnp.where(cur == mk, iota, EXPERTS), axis=-1,
                          keepdims=True)
            sels.append(idx)
            ws.append(mk)
            cur = jnp.where(iota == idx, -1.0, cur)
        sel = jnp.concatenate(sels, axis=-1)
        w = jnp.concatenate(ws, axis=-1)
        w = w / jnp.sum(w, axis=-1, keepdims=True)
        wts_ref[...] = w
        wts_scr[pl.ds(i * RB, RB), :] = w
        sel_scr[pl.ds(i * RB, RB), :] = sel
        masks = _masks_of(sel)
        onehot = jnp.zeros((RB, EXPERTS), jnp.float32)
        for k in range(K):
            onehot = onehot + masks[k].astype(jnp.float32)
        r = lax.broadcasted_iota(jnp.int32, (RB, RB), 0)
        c = lax.broadcasted_iota(jnp.int32, (RB, RB), 1)
        tri = (r > c).astype(jnp.float32)
        excl = lax.dot_general(tri, onehot, (((1,), (0,)), ((), ())),
                               preferred_element_type=jnp.float32)
        base = base_ref[...]
        vals = excl + base
        cols = [jnp.sum(jnp.where(masks[k], vals, 0.0), axis=1, keepdims=True)
                for k in range(K)]
        rank_ref[pl.ds(i * RB, RB), :] = jnp.concatenate(cols, axis=-1)
        base_ref[...] = base + jnp.sum(onehot, axis=0, keepdims=True)

    @pl.when(p == 1)
    def _pass1():
        sel = sel_scr[pl.ds(i * RB, RB), :]
        masks = _masks_of(sel)
        wts_ref[...] = wts_scr[pl.ds(i * RB, RB), :]
        counts = base_ref[...]
        ps = jnp.floor((counts + (TM - 1)) * (1.0 / TM)) * TM
        r64 = lax.broadcasted_iota(jnp.int32, (EXPERTS, EXPERTS), 0)
        c64 = lax.broadcasted_iota(jnp.int32, (EXPERTS, EXPERTS), 1)
        triu = (r64 <= c64).astype(jnp.float32)
        cum = lax.dot_general(ps, triu, (((1,), (0,)), ((), ())),
                              preferred_element_type=jnp.float32)
        pad_off = cum - ps
        rank_blk = rank_ref[pl.ds(i * RB, RB), :]
        offs = [jnp.sum(jnp.where(masks[k], pad_off, 0.0), axis=1,
                        keepdims=True) for k in range(K)]
        ppos_ref[...] = (rank_blk
                         + jnp.concatenate(offs, axis=-1)).astype(jnp.int32)

        @pl.when(i == 0)
        def _tiles():
            iota_e = lax.broadcasted_iota(jnp.int32, (1, EXPERTS), 1)
            cum_last = jnp.sum(jnp.where(iota_e == EXPERTS - 1, cum, 0.0),
                               axis=1, keepdims=True)
            starts = lax.broadcasted_iota(
                jnp.int32, (n_tiles, EXPERTS), 0).astype(
                    jnp.float32) * TM
            te = jnp.sum((cum <= starts).astype(jnp.float32), axis=1,
                         keepdims=True)
            te = jnp.minimum(te, EXPERTS - 1)
            tile_i = lax.broadcasted_iota(
                jnp.int32, (n_tiles, 1), 0).astype(jnp.float32)
            tv = (tile_i * TM < cum_last).astype(jnp.float32)
            n_valid = cum_last * (1.0 / TM)
            ti = jnp.minimum(tile_i, n_valid - 1.0)
            tiles_ref[...] = jnp.concatenate([te, tv, ti],
                                             axis=-1).astype(jnp.int32)


def _binning(flat, Wr, n_tiles):
    t = flat.shape[0]
    body = lambda *a: _binning_body(*a, n_tiles)
    return pl.pallas_call(
        body,
        grid=(2, t // RB),
        in_specs=[pl.BlockSpec((RB, DM), lambda p, i: (i, 0)),
                  pl.BlockSpec((EXPERTS, DM), lambda p, i: (0, 0))],
        out_specs=[pl.BlockSpec((RB, K), lambda p, i: (i, 0)),
                   pl.BlockSpec((RB, K), lambda p, i: (i, 0)),
                   pl.BlockSpec((n_tiles, 3), lambda p, i: (0, 0))],
        out_shape=[jax.ShapeDtypeStruct((t, K), jnp.float32),
                   jax.ShapeDtypeStruct((t, K), jnp.int32),
                   jax.ShapeDtypeStruct((n_tiles, 3), jnp.int32)],
        scratch_shapes=[pltpu.VMEM((1, EXPERTS), jnp.float32),
                        pltpu.VMEM((t, K), jnp.int32),
                        pltpu.VMEM((t, K), jnp.float32),
                        pltpu.VMEM((t, K), jnp.float32)],
        compiler_params=pltpu.CompilerParams(
            dimension_semantics=("arbitrary", "arbitrary")),
    )(flat, Wr)


def _gmm_body(meta_ref, x_ref, wg_ref, wu_ref, wd_ref, w_ref, o_ref):
    i = pl.program_id(0)

    @pl.when(meta_ref[i, 1] > 0)
    def _():
        x = x_ref[...]
        g = lax.dot_general(x, wg_ref[0].astype(jnp.bfloat16),
                            (((1,), (1,)), ((), ())),
                            preferred_element_type=jnp.float32)
        u = lax.dot_general(x, wu_ref[0].astype(jnp.bfloat16),
                            (((1,), (1,)), ((), ())),
                            preferred_element_type=jnp.float32)
        h = (g * lax.logistic(g)) * u
        o = lax.dot_general(h.astype(jnp.bfloat16),
                            wd_ref[0].astype(jnp.bfloat16),
                            (((1,), (1,)), ((), ())),
                            preferred_element_type=jnp.float32)
        o_ref[...] = (o * w_ref[...]).astype(jnp.bfloat16)


def _gmm(xs, Wg, Wu, Wd, w_padded, tiles, n_tiles):
    m_max = n_tiles * TM
    grid_spec = pltpu.PrefetchScalarGridSpec(
        num_scalar_prefetch=1,
        grid=(n_tiles,),
        in_specs=[
            pl.BlockSpec((TM, DM), lambda i, mt: (mt[i, 2], 0)),
            pl.BlockSpec((1, DH, DM), lambda i, mt: (mt[i, 0], 0, 0)),
            pl.BlockSpec((1, DH, DM), lambda i, mt: (mt[i, 0], 0, 0)),
            pl.BlockSpec((1, DM, DH), lambda i, mt: (mt[i, 0], 0, 0)),
            pl.BlockSpec((TM, 1), lambda i, mt: (mt[i, 2], 0)),
        ],
        out_specs=pl.BlockSpec((TM, DM), lambda i, mt: (mt[i, 2], 0)),
    )
    return pl.pallas_call(
        _gmm_body,
        grid_spec=grid_spec,
        out_shape=jax.ShapeDtypeStruct((m_max, DM), jnp.bfloat16),
        compiler_params=pltpu.CompilerParams(
            dimension_semantics=("arbitrary",)),
    )(tiles, xs, Wg, Wu, Wd, w_padded.reshape(m_max, 1))


def kernel(hidden_states, Wr, Wg, Wu, Wd):
    orig_shape = hidden_states.shape
    flat = hidden_states.reshape(-1, orig_shape[-1])
    t = flat.shape[0]
    a = t * K
    n_tiles = a // TM + EXPERTS
    m_max = n_tiles * TM

    wts, ppos, tiles = _binning(flat, Wr, n_tiles)
    ppos_flat = ppos.reshape(-1)
    w_padded = jnp.zeros((m_max,), jnp.float32).at[ppos_flat].set(
        wts.reshape(-1))
    tok_ids = jnp.broadcast_to(
        jnp.arange(t, dtype=jnp.int32)[:, None], (t, K)).reshape(-1)
    tok_padded = jnp.zeros((m_max,), jnp.int32).at[ppos_flat].set(tok_ids)
    xs = jnp.take(flat.astype(jnp.bfloat16), tok_padded, axis=0)

    out_s = _gmm(xs, Wg, Wu, Wd, w_padded, tiles, n_tiles)
    out = jnp.sum(out_s[ppos], axis=1, dtype=jnp.float32)
    return out.reshape(orig_shape)


# speedup vs baseline: 1.0738x; 1.0600x over previous
import jax
import jax.numpy as jnp
from jax import lax
from jax.experimental import pallas as pl
from jax.experimental.pallas import tpu as pltpu

EXPERTS = 64
K = 8
DM = 2048
DH = 768
TM = 256
TB = 256


RB = 256


def _masks_of(sel):
    iota_e = lax.broadcasted_iota(jnp.int32, (sel.shape[0], EXPERTS), 1)
    return [(sel[:, k:k + 1] == iota_e) for k in range(K)]


def _binning_body(x_ref, wr_ref, wts_ref, ppos_ref, tiles_ref,
                  base_ref, sel_scr, wts_scr, rank_ref, n_tiles):
    p = pl.program_id(0)
    i = pl.program_id(1)

    @pl.when(p == 0)
    def _pass0():
        @pl.when(i == 0)
        def _():
            base_ref[...] = jnp.zeros_like(base_ref)
        x = x_ref[...]
        logits = lax.dot_general(x, wr_ref[...], (((1,), (1,)), ((), ())),
                                 preferred_element_type=jnp.float32)
        m = jnp.max(logits, axis=-1, keepdims=True)
        prob = jnp.exp(logits - m)
        prob = prob / jnp.sum(prob, axis=-1, keepdims=True)
        iota = lax.broadcasted_iota(jnp.int32, prob.shape, 1)
        cur = prob
        sels, ws = [], []
        for _ in range(K):
            mk = jnp.max(cur, axis=-1, keepdims=True)
            idx = jnp.min(jnp.where(cur == mk, iota, EXPERTS), axis=-1,
                          keepdims=True)
            sels.append(idx)
            ws.append(mk)
            cur = jnp.where(iota == idx, -1.0, cur)
        sel = jnp.concatenate(sels, axis=-1)
        w = jnp.concatenate(ws, axis=-1)
        w = w / jnp.sum(w, axis=-1, keepdims=True)
        wts_ref[...] = w
        wts_scr[pl.ds(i * RB, RB), :] = w
        sel_scr[pl.ds(i * RB, RB), :] = sel
        masks = _masks_of(sel)
        onehot = jnp.zeros((RB, EXPERTS), jnp.float32)
        for k in range(K):
            onehot = onehot + masks[k].astype(jnp.float32)
        r = lax.broadcasted_iota(jnp.int32, (RB, RB), 0)
        c = lax.broadcasted_iota(jnp.int32, (RB, RB), 1)
        tri = (r > c).astype(jnp.float32)
        excl = lax.dot_general(tri, onehot, (((1,), (0,)), ((), ())),
                               preferred_element_type=jnp.float32)
        base = base_ref[...]
        vals = excl + base
        cols = [jnp.sum(jnp.where(masks[k], vals, 0.0), axis=1, keepdims=True)
                for k in range(K)]
        rank_ref[pl.ds(i * RB, RB), :] = jnp.concatenate(cols, axis=-1)
        base_ref[...] = base + jnp.sum(onehot, axis=0, keepdims=True)

    @pl.when(p == 1)
    def _pass1():
        sel = sel_scr[pl.ds(i * RB, RB), :]
        masks = _masks_of(sel)
        wts_ref[...] = wts_scr[pl.ds(i * RB, RB), :]
        counts = base_ref[...]
        ps = jnp.floor((counts + (TM - 1)) * (1.0 / TM)) * TM
        r64 = lax.broadcasted_iota(jnp.int32, (EXPERTS, EXPERTS), 0)
        c64 = lax.broadcasted_iota(jnp.int32, (EXPERTS, EXPERTS), 1)
        triu = (r64 <= c64).astype(jnp.float32)
        cum = lax.dot_general(ps, triu, (((1,), (0,)), ((), ())),
                              preferred_element_type=jnp.float32)
        pad_off = cum - ps
        rank_blk = rank_ref[pl.ds(i * RB, RB), :]
        offs = [jnp.sum(jnp.where(masks[k], pad_off, 0.0), axis=1,
                        keepdims=True) for k in range(K)]
        ppos_ref[...] = (rank_blk
                         + jnp.concatenate(offs, axis=-1)).astype(jnp.int32)

        @pl.when(i == 0)
        def _tiles():
            iota_e = lax.broadcasted_iota(jnp.int32, (1, EXPERTS), 1)
            cum_last = jnp.sum(jnp.where(iota_e == EXPERTS - 1, cum, 0.0),
                               axis=1, keepdims=True)
            starts = lax.broadcasted_iota(
                jnp.int32, (n_tiles, EXPERTS), 0).astype(
                    jnp.float32) * TM
            te = jnp.sum((cum <= starts).astype(jnp.float32), axis=1,
                         keepdims=True)
            te = jnp.minimum(te, EXPERTS - 1)
            tile_i = lax.broadcasted_iota(
                jnp.int32, (n_tiles, 1), 0).astype(jnp.float32)
            tv = (tile_i * TM < cum_last).astype(jnp.float32)
            n_valid = cum_last * (1.0 / TM)
            ti = jnp.minimum(tile_i, n_valid - 1.0)
            tiles_ref[...] = jnp.concatenate([te, tv, ti],
                                             axis=-1).astype(jnp.int32)


def _binning(flat, Wr, n_tiles):
    t = flat.shape[0]
    body = lambda *a: _binning_body(*a, n_tiles)
    return pl.pallas_call(
        body,
        grid=(2, t // RB),
        in_specs=[pl.BlockSpec((RB, DM), lambda p, i: (i, 0)),
                  pl.BlockSpec((EXPERTS, DM), lambda p, i: (0, 0))],
        out_specs=[pl.BlockSpec((RB, K), lambda p, i: (i, 0)),
                   pl.BlockSpec((RB, K), lambda p, i: (i, 0)),
                   pl.BlockSpec((n_tiles, 3), lambda p, i: (0, 0))],
        out_shape=[jax.ShapeDtypeStruct((t, K), jnp.float32),
                   jax.ShapeDtypeStruct((t, K), jnp.int32),
                   jax.ShapeDtypeStruct((n_tiles, 3), jnp.int32)],
        scratch_shapes=[pltpu.VMEM((1, EXPERTS), jnp.float32),
                        pltpu.VMEM((t, K), jnp.int32),
                        pltpu.VMEM((t, K), jnp.float32),
                        pltpu.VMEM((t, K), jnp.float32)],
        compiler_params=pltpu.CompilerParams(
            dimension_semantics=("arbitrary", "arbitrary")),
    )(flat, Wr)


def _gmm_body(meta_ref, x_ref, wg_ref, wu_ref, wd_ref, w_ref, o_ref):
    i = pl.program_id(0)

    @pl.when(meta_ref[i, 1] > 0)
    def _():
        x = x_ref[...]
        g = lax.dot_general(x, wg_ref[0].astype(jnp.bfloat16),
                            (((1,), (1,)), ((), ())),
                            preferred_element_type=jnp.float32)
        u = lax.dot_general(x, wu_ref[0].astype(jnp.bfloat16),
                            (((1,), (1,)), ((), ())),
                            preferred_element_type=jnp.float32)
        h = (g * lax.logistic(g)) * u
        o = lax.dot_general(h.astype(jnp.bfloat16),
                            wd_ref[0].astype(jnp.bfloat16),
                            (((1,), (1,)), ((), ())),
                            preferred_element_type=jnp.float32)
        w = lax.bitcast_convert_type(w_ref[:, 1:2], jnp.float32)
        o_ref[...] = (o * w).astype(jnp.bfloat16)


def _gmm(xs, Wg, Wu, Wd, packed, tiles, n_tiles):
    m_max = n_tiles * TM
    grid_spec = pltpu.PrefetchScalarGridSpec(
        num_scalar_prefetch=1,
        grid=(n_tiles,),
        in_specs=[
            pl.BlockSpec((TM, DM), lambda i, mt: (mt[i, 2], 0)),
            pl.BlockSpec((1, DH, DM), lambda i, mt: (mt[i, 0], 0, 0)),
            pl.BlockSpec((1, DH, DM), lambda i, mt: (mt[i, 0], 0, 0)),
            pl.BlockSpec((1, DM, DH), lambda i, mt: (mt[i, 0], 0, 0)),
            pl.BlockSpec((TM, 2), lambda i, mt: (mt[i, 2], 0)),
        ],
        out_specs=pl.BlockSpec((TM, DM), lambda i, mt: (mt[i, 2], 0)),
    )
    return pl.pallas_call(
        _gmm_body,
        grid_spec=grid_spec,
        out_shape=jax.ShapeDtypeStruct((m_max, DM), jnp.bfloat16),
        compiler_params=pltpu.CompilerParams(
            dimension_semantics=("arbitrary",)),
    )(tiles, xs, Wg, Wu, Wd, packed)


def kernel(hidden_states, Wr, Wg, Wu, Wd):
    orig_shape = hidden_states.shape
    flat = hidden_states.reshape(-1, orig_shape[-1])
    t = flat.shape[0]
    a = t * K
    n_tiles = a // TM + EXPERTS
    m_max = n_tiles * TM

    wts, ppos, tiles = _binning(flat, Wr, n_tiles)
    ppos_flat = ppos.reshape(-1)
    tok_ids = jnp.broadcast_to(
        jnp.arange(t, dtype=jnp.int32)[:, None], (t, K)).reshape(-1)
    vals = jnp.stack(
        [tok_ids, lax.bitcast_convert_type(wts, jnp.int32).reshape(-1)],
        axis=-1)
    packed = jnp.zeros((m_max, 2), jnp.int32).at[ppos_flat].set(vals)
    xs = jnp.take(flat.astype(jnp.bfloat16), packed[:, 0], axis=0)

    out_s = _gmm(xs, Wg, Wu, Wd, packed, tiles, n_tiles)
    out = jnp.sum(out_s[ppos], axis=1, dtype=jnp.float32)
    return out.reshape(orig_shape)
